# Initial kernel scaffold; baseline (speedup 1.0000x reference)
import jax
import jax.numpy as jnp
from jax.experimental import pallas as pl


def kernel(predict, target):
    raise NotImplementedError("write your pallas kernel here")


# speedup vs baseline: 4.4737x; 4.4737x over previous
import jax
import jax.numpy as jnp
from jax.experimental import pallas as pl

_B, _H, _W = 8, 512, 512
_N = _H * _W


def _bce_kernel(p_ref, t_ref, out_ref):
    b = pl.program_id(0)
    p = p_ref[0]
    t = t_ref[0]
    sel = jnp.where(t == 1.0, p, 1.0 - p)
    v = jnp.minimum(-jnp.log(sel), 100.0)
    total = jnp.sum(v)
    s1 = jnp.sum(t * v)
    c1 = jnp.sum(t)
    s0 = total - s1
    contrib = (c1 * s1 + (_N - c1) * s0) * (1.0 / (_N * float(_N) * _B))

    @pl.when(b == 0)
    def _init():
        out_ref[:, :] = jnp.zeros((1, 1), jnp.float32)

    out_ref[:, :] += jnp.full((1, 1), contrib)


def kernel(predict, target):
    out = pl.pallas_call(
        _bce_kernel,
        grid=(_B,),
        in_specs=[
            pl.BlockSpec((1, _H, _W), lambda b: (b, 0, 0)),
            pl.BlockSpec((1, _H, _W), lambda b: (b, 0, 0)),
        ],
        out_specs=pl.BlockSpec((1, 1), lambda b: (0, 0)),
        out_shape=jax.ShapeDtypeStruct((1, 1), jnp.float32),
    )(predict, target)
    return out[0, 0]
